# baseline (speedup 1.0000x reference)
import functools

import jax
import jax.numpy as jnp
import numpy as np
from jax import lax
from jax.experimental import pallas as pl
from jax.experimental.pallas import tpu as pltpu
from jax.experimental.pallas import tpu_sc as plsc

N = 16384
W = 2048
NC, NS = 2, 16
NW = NC * NS
ROWS_PER_W = N // NW
CH = 8
NCH = ROWS_PER_W // CH
NSETS = 4

SPAN_N = 1024
SPAN_P_LO = 1408
SPAN_P = W - SPAN_P_LO
ID_LO, ID_HI = 1024, 1408


def _build_indices():
    rng = np.random.RandomState(0)
    perm_p = rng.permutation(N)
    x_p = int(rng.randint(W))
    perm_n = rng.permutation(N)
    x_n = int(rng.randint(W))
    x1p = int(np.clip(x_p - 300, 0, W))
    x2p = int(np.clip(x_p + 300, 0, W))
    x1n = int(np.clip(x_n - 500, 0, W))
    x2n = int(np.clip(x_n + 500, 0, W))
    assert (x1n, x2n, x1p, x2p) == (0, 898, 1463, 2048), "unexpected windows"
    return (perm_n.astype(np.int32).reshape(NW, NCH, CH),
            perm_p.astype(np.int32).reshape(NW, NCH, CH))


_IDXN, _IDXP = _build_indices()

_SCRATCH = (
    [pltpu.VMEM((NCH, CH), jnp.int32)] * 2
    + [pltpu.VMEM((CH, W), jnp.float32)] * NSETS
    + [pltpu.VMEM((CH, 128), jnp.float32)] * NSETS
    + [pltpu.VMEM((CH, 128), jnp.float32)] * NSETS
    + [pltpu.SemaphoreType.DMA] * NSETS
    + [pltpu.SemaphoreType.DMA] * NSETS
)


@functools.partial(
    pl.kernel,
    mesh=plsc.VectorSubcoreMesh(core_axis_name="c", subcore_axis_name="s"),
    out_type=jax.ShapeDtypeStruct((N, W), jnp.float32),
    scratch_types=_SCRATCH,
)
def _gather_kernel(q, idxn, idxp, out, idxn_v, idxp_v, *bufs_and_sems):
    bufs = bufs_and_sems[0:NSETS]
    b7s = bufs_and_sems[NSETS:2 * NSETS]
    b11s = bufs_and_sems[2 * NSETS:3 * NSETS]
    gsems = bufs_and_sems[3 * NSETS:4 * NSETS]
    ssems = bufs_and_sems[4 * NSETS:5 * NSETS]

    wid = lax.axis_index("s") * NC + lax.axis_index("c")
    lane = lax.iota(jnp.int32, 16)
    m_n = lane >= 2
    m_p = lane < 7

    pltpu.sync_copy(idxn.at[wid], idxn_v)
    pltpu.sync_copy(idxp.at[wid], idxp_v)

    def issue(c, s):
        rbase = wid * ROWS_PER_W + c * CH
        buf, sem = bufs[s], gsems[s]
        pltpu.async_copy(q.at[idxn_v.at[c], pl.ds(0, SPAN_N)],
                         buf.at[:, pl.ds(0, SPAN_N)], sem)
        pltpu.async_copy(q.at[idxp_v.at[c], pl.ds(SPAN_P_LO, SPAN_P)],
                         buf.at[:, pl.ds(SPAN_P_LO, SPAN_P)], sem)
        pltpu.async_copy(q.at[pl.ds(rbase, CH), pl.ds(ID_LO, ID_HI - ID_LO)],
                         buf.at[:, pl.ds(ID_LO, ID_HI - ID_LO)], sem)
        pltpu.async_copy(q.at[pl.ds(rbase, CH), pl.ds(896, 128)], b7s[s], sem)
        pltpu.async_copy(q.at[pl.ds(rbase, CH), pl.ds(1408, 128)], b11s[s], sem)

    def drain_gather(s):
        buf, sem = bufs[s], gsems[s]
        pltpu.make_async_copy(q.at[pl.ds(0, CH), pl.ds(0, SPAN_N)],
                              buf.at[:, pl.ds(0, SPAN_N)], sem).wait()
        pltpu.make_async_copy(q.at[pl.ds(0, CH), pl.ds(0, SPAN_P)],
                              buf.at[:, pl.ds(SPAN_P_LO, SPAN_P)], sem).wait()
        pltpu.make_async_copy(q.at[pl.ds(0, CH), pl.ds(0, ID_HI - ID_LO)],
                              buf.at[:, pl.ds(ID_LO, ID_HI - ID_LO)], sem).wait()
        pltpu.make_async_copy(q.at[pl.ds(0, CH), pl.ds(0, 128)], b7s[s], sem).wait()
        pltpu.make_async_copy(q.at[pl.ds(0, CH), pl.ds(0, 128)], b11s[s], sem).wait()

    def blend(s):
        buf, b7, b11 = bufs[s], b7s[s], b11s[s]
        for r in range(CH):
            v = jnp.where(m_n, b7[r, pl.ds(0, 16)], buf[r, pl.ds(896, 16)])
            buf[r, pl.ds(896, 16)] = v
            for t in range(1, 8):
                buf[r, pl.ds(896 + 16 * t, 16)] = b7[r, pl.ds(16 * t, 16)]
            for t in range(3):
                buf[r, pl.ds(1408 + 16 * t, 16)] = b11[r, pl.ds(16 * t, 16)]
            v = jnp.where(m_p, b11[r, pl.ds(48, 16)], buf[r, pl.ds(1456, 16)])
            buf[r, pl.ds(1456, 16)] = v

    def store(c, s):
        rbase = wid * ROWS_PER_W + c * CH
        pltpu.async_copy(bufs[s], out.at[pl.ds(rbase, CH)], ssems[s])

    def wait_store(s):
        pltpu.make_async_copy(q.at[pl.ds(0, CH)], bufs[s], ssems[s]).wait()

    def process(c, s):
        drain_gather(s)
        blend(s)
        store(c, s)

    issue(0, 0)
    issue(1, 1)
    issue(2, 2)
    process(0, 0)
    issue(3, 3)
    process(1, 1)
    wait_store(0)
    issue(4, 0)
    process(2, 2)
    wait_store(1)
    issue(5, 1)
    process(3, 3)

    def body(b, carry):
        c0 = 4 * b
        for k in range(4):
            c = c0 + k
            s2 = (k + 2) % 4
            if k < 2:
                wait_store(s2)
                issue(c + 2, s2)
            else:
                @pl.when(b < NCH // 4 - 1)
                def _():
                    wait_store(s2)
                    issue(c + 2, s2)
            process(c, k)
        return carry

    lax.fori_loop(1, NCH // 4, body, 0)

    for s in range(NSETS):
        wait_store(s)


def kernel(query):
    out = _gather_kernel(query, jnp.asarray(_IDXN), jnp.asarray(_IDXP))
    return (out, out, out)
